# baseline (speedup 1.0000x reference)
import jax
import jax.numpy as jnp
from jax import lax
from jax.experimental import pallas as pl
from jax.experimental.pallas import tpu as pltpu
from jax.experimental.pallas import tpu_sc as plsc

N_NODES = 10000
N_EDGES = 320000
ATOM_DIM = 128
OUT_DIM = 128
BOND_DIM = 16
DEPTH = 6

NC, NS = 2, 16
NW = NC * NS
EW = N_EDGES // NW
CH = 200
HALF = CH // 2
NCH = EW // CH
G3 = N_EDGES // CH
NPT = 624
NTAIL = N_NODES - NPT * NS

def _sc_mesh():
    return plsc.VectorSubcoreMesh(
        core_axis_name="c", subcore_axis_name="s", num_cores=NC, num_subcores=NS)


def _worker_id():
    return lax.axis_index("s") * NC + lax.axis_index("c")


NT = NCH // 2


def _sc_gather(table, idx3):
    nrow = table.shape[0]

    dt = table.dtype

    def body(table_ref, idx3_ref, out_ref, idxbuf, rowbuf, gs0, gs1, os0, os1):
        wid = _worker_id()
        pltpu.sync_copy(idx3_ref.at[pl.ds(wid * NCH, NCH)], idxbuf)

        def issue_gather(ci, b, sem):
            pltpu.async_copy(table_ref.at[idxbuf.at[ci, 0]],
                             rowbuf.at[b, pl.ds(0, HALF)], sem)
            pltpu.async_copy(table_ref.at[idxbuf.at[ci, 1]],
                             rowbuf.at[b, pl.ds(HALF, HALF)], sem)

        def wait_gather(b, sem):
            pltpu.make_async_copy(table_ref.at[pl.ds(0, CH)],
                                  rowbuf.at[b], sem).wait()

        def wait_store(b, sem):
            pltpu.make_async_copy(rowbuf.at[b],
                                  out_ref.at[pl.ds(0, CH)], sem).wait()

        issue_gather(0, 0, gs0)
        issue_gather(1, 1, gs1)

        def step(t, carry):
            c0 = 2 * t
            wait_gather(0, gs0)
            pltpu.async_copy(rowbuf.at[0],
                             out_ref.at[pl.ds((wid * NCH + c0) * CH, CH)], os0)
            wait_gather(1, gs1)
            pltpu.async_copy(rowbuf.at[1],
                             out_ref.at[pl.ds((wid * NCH + c0 + 1) * CH, CH)],
                             os1)

            @pl.when(t + 1 < NT)
            def _next():
                wait_store(0, os0)
                issue_gather(c0 + 2, 0, gs0)
                wait_store(1, os1)
                issue_gather(c0 + 3, 1, gs1)

            return carry

        lax.fori_loop(0, NT, step, 0)
        wait_store(0, os0)
        wait_store(1, os1)

    f = pl.kernel(
        body,
        out_type=jax.ShapeDtypeStruct((N_EDGES, OUT_DIM), dt),
        mesh=_sc_mesh(),
        scratch_types=[
            pltpu.VMEM((NCH, 2, HALF), jnp.int32),
            pltpu.VMEM((2, CH, OUT_DIM), dt),
            pltpu.SemaphoreType.DMA,
            pltpu.SemaphoreType.DMA,
            pltpu.SemaphoreType.DMA,
            pltpu.SemaphoreType.DMA,
        ],
        name=f"sc_gather_{nrow}",
    )
    return f(table, idx3)


CHS = 80
NCHS = EW // CHS
NTS = (NCHS + 1) // 2


def _sc_scatter(rows, idxw, zeros):
    def body(rows_ref, idxw_ref, zeros_ref, parts_ref, idxbuf, rowbuf, esum,
             ls0, ls1):
        cid = lax.axis_index("c")
        sid = lax.axis_index("s")
        wid = _worker_id()

        pltpu.sync_copy(idxw_ref.at[wid], idxbuf)

        def issue_load(ci, b, sem):
            pltpu.async_copy(rows_ref.at[pl.ds((wid * NCHS + ci) * CHS, CHS)],
                             rowbuf.at[b], sem)

        def wait_load(b, sem):
            pltpu.make_async_copy(rows_ref.at[pl.ds(0, CHS)],
                                  rowbuf.at[b], sem).wait()

        def scatter(ci, b):
            pltpu.sync_copy(rowbuf.at[b], esum.at[idxbuf.at[ci]], add=True)

        issue_load(0, 0, ls0)
        issue_load(1, 1, ls1)

        pltpu.sync_copy(zeros_ref.at[pl.ds(sid * NPT, NPT)],
                        esum.at[pl.ds(sid * NPT, NPT)])

        @pl.when(sid == 0)
        def _init_tail():
            pltpu.sync_copy(zeros_ref.at[pl.ds(NPT * NS, NTAIL)],
                            esum.at[pl.ds(NPT * NS, NTAIL)])

        plsc.subcore_barrier()

        def step(t, carry):
            c0 = 2 * t
            wait_load(0, ls0)
            scatter(c0, 0)

            @pl.when(c0 + 2 < NCHS)
            def _next0():
                issue_load(c0 + 2, 0, ls0)

            @pl.when(c0 + 1 < NCHS)
            def _odd():
                wait_load(1, ls1)
                scatter(c0 + 1, 1)

                @pl.when(c0 + 3 < NCHS)
                def _next1():
                    issue_load(c0 + 3, 1, ls1)

            return carry

        lax.fori_loop(0, NTS, step, 0)
        plsc.subcore_barrier()
        pltpu.sync_copy(esum.at[pl.ds(sid * NPT, NPT)],
                        parts_ref.at[cid, pl.ds(sid * NPT, NPT)])

        @pl.when(sid == 0)
        def _out_tail():
            pltpu.sync_copy(esum.at[pl.ds(NPT * NS, NTAIL)],
                            parts_ref.at[cid, pl.ds(NPT * NS, NTAIL)])

    f = pl.kernel(
        body,
        out_type=jax.ShapeDtypeStruct((NC, N_NODES, OUT_DIM), jnp.float32),
        mesh=_sc_mesh(),
        scratch_types=[
            pltpu.VMEM((NCHS, CHS), jnp.int32),
            pltpu.VMEM((2, CHS, OUT_DIM), jnp.float32),
            pltpu.VMEM_SHARED((N_NODES, OUT_DIM), jnp.float32),
            pltpu.SemaphoreType.DMA,
            pltpu.SemaphoreType.DMA,
        ],
        name="sc_scatter_add",
    )
    return f(rows, idxw, zeros)


_EB = 512
_EG = N_EDGES // _EB
_NB = 1000
_NG = N_NODES // _NB


def _p0_body(xg_ref, ea_ref, wx_ref, wb_ref, inp_ref, msg_ref):
    v = jnp.dot(xg_ref[...], wx_ref[...], preferred_element_type=jnp.float32)
    v += jnp.dot(ea_ref[...], wb_ref[...], preferred_element_type=jnp.float32)
    inp_ref[...] = v
    msg_ref[...] = jnp.maximum(v, 0.0)


def _tc_p0(xg, ea, wxt, wbt):
    return pl.pallas_call(
        _p0_body,
        grid=(_EG,),
        in_specs=[
            pl.BlockSpec((_EB, ATOM_DIM), lambda i: (i, 0)),
            pl.BlockSpec((_EB, BOND_DIM), lambda i: (i, 0)),
            pl.BlockSpec((ATOM_DIM, OUT_DIM), lambda i: (0, 0)),
            pl.BlockSpec((BOND_DIM, OUT_DIM), lambda i: (0, 0)),
        ],
        out_specs=[
            pl.BlockSpec((_EB, OUT_DIM), lambda i: (i, 0)),
            pl.BlockSpec((_EB, OUT_DIM), lambda i: (i, 0)),
        ],
        out_shape=[
            jax.ShapeDtypeStruct((N_EDGES, OUT_DIM), jnp.float32),
            jax.ShapeDtypeStruct((N_EDGES, OUT_DIM), jnp.float32),
        ],
    )(xg, ea, wxt, wbt)


def _swap_pairs(a3):
    return jnp.concatenate([a3[:, 1:2, :], a3[:, 0:1, :]], axis=1)


def _depth_body(inp_ref, msg3_ref, gm_ref, wh_ref, out_ref):
    msw = _swap_pairs(msg3_ref[...]).reshape(_EB, OUT_DIM)
    t = jnp.dot(msw, wh_ref[...], preferred_element_type=jnp.float32)
    out_ref[...] = jnp.maximum(inp_ref[...] + gm_ref[...] - t, 0.0)


def _tc_depth(inp, msg3, gm, wht):
    return pl.pallas_call(
        _depth_body,
        grid=(_EG,),
        in_specs=[
            pl.BlockSpec((_EB, OUT_DIM), lambda i: (i, 0)),
            pl.BlockSpec((_EB // 2, 2, OUT_DIM), lambda i: (i, 0, 0)),
            pl.BlockSpec((_EB, OUT_DIM), lambda i: (i, 0)),
            pl.BlockSpec((OUT_DIM, OUT_DIM), lambda i: (0, 0)),
        ],
        out_specs=pl.BlockSpec((_EB, OUT_DIM), lambda i: (i, 0)),
        out_shape=jax.ShapeDtypeStruct((N_EDGES, OUT_DIM), jnp.float32),
    )(inp, msg3, gm, wht)


def _g_body(parts_ref, wh_ref, out_ref):
    p = parts_ref[...]
    out_ref[...] = jnp.dot(p[0] + p[1], wh_ref[...],
                           preferred_element_type=jnp.float32)


def _tc_g(parts, wht):
    return pl.pallas_call(
        _g_body,
        grid=(_NG,),
        in_specs=[
            pl.BlockSpec((NC, _NB, OUT_DIM), lambda i: (0, i, 0)),
            pl.BlockSpec((OUT_DIM, OUT_DIM), lambda i: (0, 0)),
        ],
        out_specs=pl.BlockSpec((_NB, OUT_DIM), lambda i: (i, 0)),
        out_shape=jax.ShapeDtypeStruct((N_NODES, OUT_DIM), jnp.float32),
    )(parts, wht)


def _final_body(x_ref, parts_ref, wox_ref, woo_ref, bo_ref, out_ref):
    p = parts_ref[...]
    h = jnp.dot(x_ref[...], wox_ref[...], preferred_element_type=jnp.float32)
    h += jnp.dot(p[0] + p[1], woo_ref[...], preferred_element_type=jnp.float32)
    out_ref[...] = jnp.maximum(h + bo_ref[...], 0.0)


def _tc_final(x, parts, woxt, woot, bo):
    return pl.pallas_call(
        _final_body,
        grid=(_NG,),
        in_specs=[
            pl.BlockSpec((_NB, ATOM_DIM), lambda i: (i, 0)),
            pl.BlockSpec((NC, _NB, OUT_DIM), lambda i: (0, i, 0)),
            pl.BlockSpec((ATOM_DIM, OUT_DIM), lambda i: (0, 0)),
            pl.BlockSpec((OUT_DIM, OUT_DIM), lambda i: (0, 0)),
            pl.BlockSpec((1, OUT_DIM), lambda i: (0, 0)),
        ],
        out_specs=pl.BlockSpec((_NB, OUT_DIM), lambda i: (i, 0)),
        out_shape=jax.ShapeDtypeStruct((N_NODES, OUT_DIM), jnp.float32),
    )(x, parts, woxt, woot, bo)


def kernel(x, edge_index, edge_attr, W_i, W_h, W_o, b_o):
    src = edge_index[0].astype(jnp.int32)
    dst = edge_index[1].astype(jnp.int32)
    src3 = src.reshape(G3, 2, HALF)
    dsw = dst.reshape(-1, 2)[:, ::-1].reshape(-1)
    dsw3 = dsw.reshape(G3, 2, HALF)
    dstw = dst.reshape(NW, NCHS, CHS)
    zeros = jnp.zeros((N_NODES, OUT_DIM), jnp.float32)

    wxt = W_i[:, :ATOM_DIM].T
    wbt = W_i[:, ATOM_DIM:].T
    wht = W_h.T
    woxt = W_o[:, :ATOM_DIM].T
    woot = W_o[:, ATOM_DIM:].T
    bo2 = b_o.reshape(1, OUT_DIM)

    xg = _sc_gather(x, src3)
    inp, msg = _tc_p0(xg, edge_attr, wxt, wbt)

    parts = _sc_scatter(msg, dstw, zeros)
    for _ in range(DEPTH - 1):
        g = _tc_g(parts, wht)
        gm = _sc_gather(g, dsw3)
        msg3 = msg.reshape(N_EDGES // 2, 2, OUT_DIM)
        msg = _tc_depth(inp, msg3, gm, wht)
        parts = _sc_scatter(msg, dstw, zeros)

    return _tc_final(x, parts, woxt, woot, bo2)


# speedup vs baseline: 1.1471x; 1.1471x over previous
import jax
import jax.numpy as jnp
from jax import lax
from jax.experimental import pallas as pl
from jax.experimental.pallas import tpu as pltpu
from jax.experimental.pallas import tpu_sc as plsc

N_NODES = 10000
N_EDGES = 320000
ATOM_DIM = 128
OUT_DIM = 128
BOND_DIM = 16
DEPTH = 6

NC, NS = 2, 16
NW = NC * NS
EW = N_EDGES // NW
CH = 200
HALF = CH // 2
NCH = EW // CH
G3 = N_EDGES // CH
NPT = 624
NTAIL = N_NODES - NPT * NS

def _sc_mesh():
    return plsc.VectorSubcoreMesh(
        core_axis_name="c", subcore_axis_name="s", num_cores=NC, num_subcores=NS)


def _worker_id():
    return lax.axis_index("s") * NC + lax.axis_index("c")


NT = NCH // 2


def _sc_gather(table, idx3):
    nrow = table.shape[0]

    dt = table.dtype

    def body(table_ref, idx3_ref, out_ref, idxbuf, rowbuf, gs0, gs1, os0, os1):
        wid = _worker_id()
        pltpu.sync_copy(idx3_ref.at[pl.ds(wid * NCH, NCH)], idxbuf)

        def issue_gather(ci, b, sem):
            pltpu.async_copy(table_ref.at[idxbuf.at[ci, 0]],
                             rowbuf.at[b, pl.ds(0, HALF)], sem)
            pltpu.async_copy(table_ref.at[idxbuf.at[ci, 1]],
                             rowbuf.at[b, pl.ds(HALF, HALF)], sem)

        def wait_gather(b, sem):
            pltpu.make_async_copy(table_ref.at[pl.ds(0, CH)],
                                  rowbuf.at[b], sem).wait()

        def wait_store(b, sem):
            pltpu.make_async_copy(rowbuf.at[b],
                                  out_ref.at[pl.ds(0, CH)], sem).wait()

        issue_gather(0, 0, gs0)
        issue_gather(1, 1, gs1)

        def step(t, carry):
            c0 = 2 * t
            wait_gather(0, gs0)
            pltpu.async_copy(rowbuf.at[0],
                             out_ref.at[pl.ds((wid * NCH + c0) * CH, CH)], os0)
            wait_gather(1, gs1)
            pltpu.async_copy(rowbuf.at[1],
                             out_ref.at[pl.ds((wid * NCH + c0 + 1) * CH, CH)],
                             os1)

            @pl.when(t + 1 < NT)
            def _next():
                wait_store(0, os0)
                issue_gather(c0 + 2, 0, gs0)
                wait_store(1, os1)
                issue_gather(c0 + 3, 1, gs1)

            return carry

        lax.fori_loop(0, NT, step, 0)
        wait_store(0, os0)
        wait_store(1, os1)

    f = pl.kernel(
        body,
        out_type=jax.ShapeDtypeStruct((N_EDGES, OUT_DIM), dt),
        mesh=_sc_mesh(),
        scratch_types=[
            pltpu.VMEM((NCH, 2, HALF), jnp.int32),
            pltpu.VMEM((2, CH, OUT_DIM), dt),
            pltpu.SemaphoreType.DMA,
            pltpu.SemaphoreType.DMA,
            pltpu.SemaphoreType.DMA,
            pltpu.SemaphoreType.DMA,
        ],
        name=f"sc_gather_{nrow}",
    )
    return f(table, idx3)


CHS = 80
NCHS = EW // CHS
NTS = (NCHS + 1) // 2


def _sc_scatter(rows, idxw, zeros):
    def body(rows_ref, idxw_ref, zeros_ref, parts_ref, idxbuf, rowbuf, esum,
             ls0, ls1):
        cid = lax.axis_index("c")
        sid = lax.axis_index("s")
        wid = _worker_id()

        pltpu.sync_copy(idxw_ref.at[wid], idxbuf)

        def issue_load(ci, b, sem):
            pltpu.async_copy(rows_ref.at[pl.ds((wid * NCHS + ci) * CHS, CHS)],
                             rowbuf.at[b], sem)

        def wait_load(b, sem):
            pltpu.make_async_copy(rows_ref.at[pl.ds(0, CHS)],
                                  rowbuf.at[b], sem).wait()

        def scatter(ci, b):
            pltpu.sync_copy(rowbuf.at[b], esum.at[idxbuf.at[ci]], add=True)

        issue_load(0, 0, ls0)
        issue_load(1, 1, ls1)

        pltpu.sync_copy(zeros_ref.at[pl.ds(sid * NPT, NPT)],
                        esum.at[pl.ds(sid * NPT, NPT)])

        @pl.when(sid == 0)
        def _init_tail():
            pltpu.sync_copy(zeros_ref.at[pl.ds(NPT * NS, NTAIL)],
                            esum.at[pl.ds(NPT * NS, NTAIL)])

        plsc.subcore_barrier()

        def step(t, carry):
            c0 = 2 * t
            wait_load(0, ls0)
            scatter(c0, 0)

            @pl.when(c0 + 2 < NCHS)
            def _next0():
                issue_load(c0 + 2, 0, ls0)

            @pl.when(c0 + 1 < NCHS)
            def _odd():
                wait_load(1, ls1)
                scatter(c0 + 1, 1)

                @pl.when(c0 + 3 < NCHS)
                def _next1():
                    issue_load(c0 + 3, 1, ls1)

            return carry

        lax.fori_loop(0, NTS, step, 0)
        plsc.subcore_barrier()
        pltpu.sync_copy(esum.at[pl.ds(sid * NPT, NPT)],
                        parts_ref.at[cid, pl.ds(sid * NPT, NPT)])

        @pl.when(sid == 0)
        def _out_tail():
            pltpu.sync_copy(esum.at[pl.ds(NPT * NS, NTAIL)],
                            parts_ref.at[cid, pl.ds(NPT * NS, NTAIL)])

    f = pl.kernel(
        body,
        out_type=jax.ShapeDtypeStruct((NC, N_NODES, OUT_DIM), jnp.float32),
        mesh=_sc_mesh(),
        scratch_types=[
            pltpu.VMEM((NCHS, CHS), jnp.int32),
            pltpu.VMEM((2, CHS, OUT_DIM), jnp.float32),
            pltpu.VMEM_SHARED((N_NODES, OUT_DIM), jnp.float32),
            pltpu.SemaphoreType.DMA,
            pltpu.SemaphoreType.DMA,
        ],
        name="sc_scatter_add",
    )
    return f(rows, idxw, zeros)


_EB = 512
_EG = N_EDGES // _EB
_NB = 1000
_NG = N_NODES // _NB


def _p0_body(xg_ref, ea_ref, wx_ref, wb_ref, inp_ref, msg_ref):
    v = jnp.dot(xg_ref[...], wx_ref[...], preferred_element_type=jnp.float32)
    v += jnp.dot(ea_ref[...], wb_ref[...], preferred_element_type=jnp.float32)
    inp_ref[...] = v.astype(inp_ref.dtype)
    msg_ref[...] = jnp.maximum(v, 0.0)


def _tc_p0(xg, ea, wxt, wbt):
    return pl.pallas_call(
        _p0_body,
        grid=(_EG,),
        in_specs=[
            pl.BlockSpec((_EB, ATOM_DIM), lambda i: (i, 0)),
            pl.BlockSpec((_EB, BOND_DIM), lambda i: (i, 0)),
            pl.BlockSpec((ATOM_DIM, OUT_DIM), lambda i: (0, 0)),
            pl.BlockSpec((BOND_DIM, OUT_DIM), lambda i: (0, 0)),
        ],
        out_specs=[
            pl.BlockSpec((_EB, OUT_DIM), lambda i: (i, 0)),
            pl.BlockSpec((_EB, OUT_DIM), lambda i: (i, 0)),
        ],
        out_shape=[
            jax.ShapeDtypeStruct((N_EDGES, OUT_DIM), jnp.bfloat16),
            jax.ShapeDtypeStruct((N_EDGES, OUT_DIM), jnp.float32),
        ],
    )(xg, ea, wxt, wbt)


def _swap_pairs(a3):
    return jnp.concatenate([a3[:, 1:2, :], a3[:, 0:1, :]], axis=1)


def _depth_body(inp_ref, msg_ref, gm_ref, wh_ref, out_ref):
    t = jnp.dot(msg_ref[...], wh_ref[...], preferred_element_type=jnp.float32)
    d3 = (gm_ref[...] - t).reshape(_EB // 2, 2, OUT_DIM)
    sw = _swap_pairs(d3).reshape(_EB, OUT_DIM)
    out_ref[...] = jnp.maximum(inp_ref[...].astype(jnp.float32) + sw, 0.0)


def _tc_depth(inp, msg, gm, wht):
    return pl.pallas_call(
        _depth_body,
        grid=(_EG,),
        in_specs=[
            pl.BlockSpec((_EB, OUT_DIM), lambda i: (i, 0)),
            pl.BlockSpec((_EB, OUT_DIM), lambda i: (i, 0)),
            pl.BlockSpec((_EB, OUT_DIM), lambda i: (i, 0)),
            pl.BlockSpec((OUT_DIM, OUT_DIM), lambda i: (0, 0)),
        ],
        out_specs=pl.BlockSpec((_EB, OUT_DIM), lambda i: (i, 0)),
        out_shape=jax.ShapeDtypeStruct((N_EDGES, OUT_DIM), jnp.float32),
    )(inp, msg, gm, wht)


def _g_body(parts_ref, wh_ref, out_ref):
    p = parts_ref[...]
    out_ref[...] = jnp.dot(p[0] + p[1], wh_ref[...],
                           preferred_element_type=jnp.float32)


def _tc_g(parts, wht):
    return pl.pallas_call(
        _g_body,
        grid=(_NG,),
        in_specs=[
            pl.BlockSpec((NC, _NB, OUT_DIM), lambda i: (0, i, 0)),
            pl.BlockSpec((OUT_DIM, OUT_DIM), lambda i: (0, 0)),
        ],
        out_specs=pl.BlockSpec((_NB, OUT_DIM), lambda i: (i, 0)),
        out_shape=jax.ShapeDtypeStruct((N_NODES, OUT_DIM), jnp.float32),
    )(parts, wht)


def _final_body(x_ref, parts_ref, wox_ref, woo_ref, bo_ref, out_ref):
    p = parts_ref[...]
    h = jnp.dot(x_ref[...], wox_ref[...], preferred_element_type=jnp.float32)
    h += jnp.dot(p[0] + p[1], woo_ref[...], preferred_element_type=jnp.float32)
    out_ref[...] = jnp.maximum(h + bo_ref[...], 0.0)


def _tc_final(x, parts, woxt, woot, bo):
    return pl.pallas_call(
        _final_body,
        grid=(_NG,),
        in_specs=[
            pl.BlockSpec((_NB, ATOM_DIM), lambda i: (i, 0)),
            pl.BlockSpec((NC, _NB, OUT_DIM), lambda i: (0, i, 0)),
            pl.BlockSpec((ATOM_DIM, OUT_DIM), lambda i: (0, 0)),
            pl.BlockSpec((OUT_DIM, OUT_DIM), lambda i: (0, 0)),
            pl.BlockSpec((1, OUT_DIM), lambda i: (0, 0)),
        ],
        out_specs=pl.BlockSpec((_NB, OUT_DIM), lambda i: (i, 0)),
        out_shape=jax.ShapeDtypeStruct((N_NODES, OUT_DIM), jnp.float32),
    )(x, parts, woxt, woot, bo)


def kernel(x, edge_index, edge_attr, W_i, W_h, W_o, b_o):
    src = edge_index[0].astype(jnp.int32)
    dst = edge_index[1].astype(jnp.int32)
    src3 = src.reshape(G3, 2, HALF)
    dst3 = dst.reshape(G3, 2, HALF)
    dstw = dst.reshape(NW, NCHS, CHS)
    zeros = jnp.zeros((N_NODES, OUT_DIM), jnp.float32)

    wxt = W_i[:, :ATOM_DIM].T
    wbt = W_i[:, ATOM_DIM:].T
    wht = W_h.T
    woxt = W_o[:, :ATOM_DIM].T
    woot = W_o[:, ATOM_DIM:].T
    bo2 = b_o.reshape(1, OUT_DIM)

    xg = _sc_gather(x, src3)
    inp, msg = _tc_p0(xg, edge_attr, wxt, wbt)

    parts = _sc_scatter(msg, dstw, zeros)
    for _ in range(DEPTH - 1):
        g = _tc_g(parts, wht)
        gm = _sc_gather(g, dst3)
        msg = _tc_depth(inp, msg, gm, wht)
        parts = _sc_scatter(msg, dstw, zeros)

    return _tc_final(x, parts, woxt, woot, bo2)


# speedup vs baseline: 1.4498x; 1.2639x over previous
import jax
import jax.numpy as jnp
from jax import lax
from jax.experimental import pallas as pl
from jax.experimental.pallas import tpu as pltpu
from jax.experimental.pallas import tpu_sc as plsc

N_NODES = 10000
N_EDGES = 320000
ATOM_DIM = 128
OUT_DIM = 128
BOND_DIM = 16
DEPTH = 6

NC, NS = 2, 16
NW = NC * NS
EW = N_EDGES // NW
CH = 200
HALF = CH // 2
NCH = EW // CH
G3 = N_EDGES // CH
NPT = 624
NTAIL = N_NODES - NPT * NS

def _sc_mesh():
    return plsc.VectorSubcoreMesh(
        core_axis_name="c", subcore_axis_name="s", num_cores=NC, num_subcores=NS)


def _worker_id():
    return lax.axis_index("s") * NC + lax.axis_index("c")


NT = NCH // 2


def _sc_gather(table, idx3):
    nrow = table.shape[0]

    dt = table.dtype

    def body(table_ref, idx3_ref, out_ref, idxbuf, rowbuf, gs0, gs1, os0, os1):
        wid = _worker_id()
        pltpu.sync_copy(idx3_ref.at[pl.ds(wid * NCH, NCH)], idxbuf)

        def issue_gather(ci, b, sem):
            pltpu.async_copy(table_ref.at[idxbuf.at[ci, 0]],
                             rowbuf.at[b, pl.ds(0, HALF)], sem)
            pltpu.async_copy(table_ref.at[idxbuf.at[ci, 1]],
                             rowbuf.at[b, pl.ds(HALF, HALF)], sem)

        def wait_gather(b, sem):
            pltpu.make_async_copy(table_ref.at[pl.ds(0, CH)],
                                  rowbuf.at[b], sem).wait()

        def wait_store(b, sem):
            pltpu.make_async_copy(rowbuf.at[b],
                                  out_ref.at[pl.ds(0, CH)], sem).wait()

        issue_gather(0, 0, gs0)
        issue_gather(1, 1, gs1)

        def step(t, carry):
            c0 = 2 * t
            wait_gather(0, gs0)
            pltpu.async_copy(rowbuf.at[0],
                             out_ref.at[pl.ds((wid * NCH + c0) * CH, CH)], os0)
            wait_gather(1, gs1)
            pltpu.async_copy(rowbuf.at[1],
                             out_ref.at[pl.ds((wid * NCH + c0 + 1) * CH, CH)],
                             os1)

            @pl.when(t + 1 < NT)
            def _next():
                wait_store(0, os0)
                issue_gather(c0 + 2, 0, gs0)
                wait_store(1, os1)
                issue_gather(c0 + 3, 1, gs1)

            return carry

        lax.fori_loop(0, NT, step, 0)
        wait_store(0, os0)
        wait_store(1, os1)

    f = pl.kernel(
        body,
        out_type=jax.ShapeDtypeStruct((N_EDGES, OUT_DIM), dt),
        mesh=_sc_mesh(),
        scratch_types=[
            pltpu.VMEM((NCH, 2, HALF), jnp.int32),
            pltpu.VMEM((2, CH, OUT_DIM), dt),
            pltpu.SemaphoreType.DMA,
            pltpu.SemaphoreType.DMA,
            pltpu.SemaphoreType.DMA,
            pltpu.SemaphoreType.DMA,
        ],
        name=f"sc_gather_{nrow}",
    )
    return f(table, idx3)


CHS = 80
NCHS = EW // CHS
NTS = (NCHS + 1) // 2


def _sc_scatter(rows, idxw, zeros):
    def body(rows_ref, idxw_ref, zeros_ref, parts_ref, idxbuf, rowbuf, esum,
             ls0, ls1):
        cid = lax.axis_index("c")
        sid = lax.axis_index("s")
        wid = _worker_id()

        pltpu.sync_copy(idxw_ref.at[wid], idxbuf)

        def issue_load(ci, b, sem):
            pltpu.async_copy(rows_ref.at[pl.ds((wid * NCHS + ci) * CHS, CHS)],
                             rowbuf.at[b], sem)

        def wait_load(b, sem):
            pltpu.make_async_copy(rows_ref.at[pl.ds(0, CHS)],
                                  rowbuf.at[b], sem).wait()

        def scatter(ci, b):
            pltpu.sync_copy(rowbuf.at[b], esum.at[idxbuf.at[ci]], add=True)

        issue_load(0, 0, ls0)
        issue_load(1, 1, ls1)

        pltpu.sync_copy(zeros_ref.at[pl.ds(sid * NPT, NPT)],
                        esum.at[pl.ds(sid * NPT, NPT)])

        @pl.when(sid == 0)
        def _init_tail():
            pltpu.sync_copy(zeros_ref.at[pl.ds(NPT * NS, NTAIL)],
                            esum.at[pl.ds(NPT * NS, NTAIL)])

        plsc.subcore_barrier()

        def step(t, carry):
            c0 = 2 * t
            wait_load(0, ls0)
            scatter(c0, 0)

            @pl.when(c0 + 2 < NCHS)
            def _next0():
                issue_load(c0 + 2, 0, ls0)

            @pl.when(c0 + 1 < NCHS)
            def _odd():
                wait_load(1, ls1)
                scatter(c0 + 1, 1)

                @pl.when(c0 + 3 < NCHS)
                def _next1():
                    issue_load(c0 + 3, 1, ls1)

            return carry

        lax.fori_loop(0, NTS, step, 0)
        plsc.subcore_barrier()
        pltpu.sync_copy(esum.at[pl.ds(sid * NPT, NPT)],
                        parts_ref.at[cid, pl.ds(sid * NPT, NPT)])

        @pl.when(sid == 0)
        def _out_tail():
            pltpu.sync_copy(esum.at[pl.ds(NPT * NS, NTAIL)],
                            parts_ref.at[cid, pl.ds(NPT * NS, NTAIL)])

    f = pl.kernel(
        body,
        out_type=jax.ShapeDtypeStruct((NC, N_NODES, OUT_DIM), jnp.float32),
        mesh=_sc_mesh(),
        scratch_types=[
            pltpu.VMEM((NCHS, CHS), jnp.int32),
            pltpu.VMEM((2, CHS, OUT_DIM), jnp.float32),
            pltpu.VMEM_SHARED((N_NODES, OUT_DIM), jnp.float32),
            pltpu.SemaphoreType.DMA,
            pltpu.SemaphoreType.DMA,
        ],
        name="sc_scatter_add",
    )
    return f(rows, idxw, zeros)


_EB = 1024
_EG = N_EDGES // _EB
_NB = 1000
_NG = N_NODES // _NB


def _p0_body(xg_ref, ea_ref, wx_ref, wb_ref, inp_ref, msg_ref):
    v = jnp.dot(xg_ref[...], wx_ref[...], preferred_element_type=jnp.float32)
    v += jnp.dot(ea_ref[...], wb_ref[...], preferred_element_type=jnp.float32)
    inp_ref[...] = v.astype(inp_ref.dtype)
    msg_ref[...] = jnp.maximum(v, 0.0)


def _tc_p0(xg, ea, wxt, wbt):
    return pl.pallas_call(
        _p0_body,
        grid=(_EG,),
        in_specs=[
            pl.BlockSpec((_EB, ATOM_DIM), lambda i: (i, 0)),
            pl.BlockSpec((_EB, BOND_DIM), lambda i: (i, 0)),
            pl.BlockSpec((ATOM_DIM, OUT_DIM), lambda i: (0, 0)),
            pl.BlockSpec((BOND_DIM, OUT_DIM), lambda i: (0, 0)),
        ],
        out_specs=[
            pl.BlockSpec((_EB, OUT_DIM), lambda i: (i, 0)),
            pl.BlockSpec((_EB, OUT_DIM), lambda i: (i, 0)),
        ],
        out_shape=[
            jax.ShapeDtypeStruct((N_EDGES, OUT_DIM), jnp.bfloat16),
            jax.ShapeDtypeStruct((N_EDGES, OUT_DIM), jnp.float32),
        ],
    )(xg, ea, wxt, wbt)


def _swap_pairs(a3):
    return jnp.concatenate([a3[:, 1:2, :], a3[:, 0:1, :]], axis=1)


def _depth_body(inp_ref, msg_ref, gm_ref, wh_ref, out_ref):
    t = jnp.dot(msg_ref[...], wh_ref[...], preferred_element_type=jnp.float32)
    d3 = (gm_ref[...] - t).reshape(_EB // 2, 2, OUT_DIM)
    sw = _swap_pairs(d3).reshape(_EB, OUT_DIM)
    out_ref[...] = jnp.maximum(inp_ref[...].astype(jnp.float32) + sw, 0.0)


def _tc_depth(inp, msg, gm, wht):
    return pl.pallas_call(
        _depth_body,
        grid=(_EG,),
        in_specs=[
            pl.BlockSpec((_EB, OUT_DIM), lambda i: (i, 0)),
            pl.BlockSpec((_EB, OUT_DIM), lambda i: (i, 0)),
            pl.BlockSpec((_EB, OUT_DIM), lambda i: (i, 0)),
            pl.BlockSpec((OUT_DIM, OUT_DIM), lambda i: (0, 0)),
        ],
        out_specs=pl.BlockSpec((_EB, OUT_DIM), lambda i: (i, 0)),
        out_shape=jax.ShapeDtypeStruct((N_EDGES, OUT_DIM), jnp.float32),
    )(inp, msg, gm, wht)


def _g_body(parts_ref, wh_ref, out_ref):
    p = parts_ref[...]
    out_ref[...] = jnp.dot(p[0] + p[1], wh_ref[...],
                           preferred_element_type=jnp.float32)


def _tc_g(parts, wht):
    return pl.pallas_call(
        _g_body,
        grid=(_NG,),
        in_specs=[
            pl.BlockSpec((NC, _NB, OUT_DIM), lambda i: (0, i, 0)),
            pl.BlockSpec((OUT_DIM, OUT_DIM), lambda i: (0, 0)),
        ],
        out_specs=pl.BlockSpec((_NB, OUT_DIM), lambda i: (i, 0)),
        out_shape=jax.ShapeDtypeStruct((N_NODES, OUT_DIM), jnp.float32),
    )(parts, wht)


def _final_body(x_ref, parts_ref, wox_ref, woo_ref, bo_ref, out_ref):
    p = parts_ref[...]
    h = jnp.dot(x_ref[...], wox_ref[...], preferred_element_type=jnp.float32)
    h += jnp.dot(p[0] + p[1], woo_ref[...], preferred_element_type=jnp.float32)
    out_ref[...] = jnp.maximum(h + bo_ref[...], 0.0)


def _tc_final(x, parts, woxt, woot, bo):
    return pl.pallas_call(
        _final_body,
        grid=(_NG,),
        in_specs=[
            pl.BlockSpec((_NB, ATOM_DIM), lambda i: (i, 0)),
            pl.BlockSpec((NC, _NB, OUT_DIM), lambda i: (0, i, 0)),
            pl.BlockSpec((ATOM_DIM, OUT_DIM), lambda i: (0, 0)),
            pl.BlockSpec((OUT_DIM, OUT_DIM), lambda i: (0, 0)),
            pl.BlockSpec((1, OUT_DIM), lambda i: (0, 0)),
        ],
        out_specs=pl.BlockSpec((_NB, OUT_DIM), lambda i: (i, 0)),
        out_shape=jax.ShapeDtypeStruct((N_NODES, OUT_DIM), jnp.float32),
    )(x, parts, woxt, woot, bo)


def kernel(x, edge_index, edge_attr, W_i, W_h, W_o, b_o):
    src = edge_index[0].astype(jnp.int32)
    dst = edge_index[1].astype(jnp.int32)
    src3 = src.reshape(G3, 2, HALF)
    dst3 = dst.reshape(G3, 2, HALF)
    dstw = dst.reshape(NW, NCHS, CHS)
    zeros = jnp.zeros((N_NODES, OUT_DIM), jnp.float32)

    wxt = W_i[:, :ATOM_DIM].T
    wbt = W_i[:, ATOM_DIM:].T
    wht = W_h.T
    woxt = W_o[:, :ATOM_DIM].T
    woot = W_o[:, ATOM_DIM:].T
    bo2 = b_o.reshape(1, OUT_DIM)

    xg = _sc_gather(x, src3)
    inp, msg = _tc_p0(xg, edge_attr, wxt, wbt)

    parts = _sc_scatter(msg, dstw, zeros)
    for _ in range(DEPTH - 1):
        g = _tc_g(parts, wht)
        gm = _sc_gather(g, dst3)
        msg = _tc_depth(inp, msg, gm, wht)
        parts = _sc_scatter(msg, dstw, zeros)

    return _tc_final(x, parts, woxt, woot, bo2)


# speedup vs baseline: 1.5309x; 1.0560x over previous
import jax
import jax.numpy as jnp
from jax import lax
from jax.experimental import pallas as pl
from jax.experimental.pallas import tpu as pltpu
from jax.experimental.pallas import tpu_sc as plsc

N_NODES = 10000
N_EDGES = 320000
ATOM_DIM = 128
OUT_DIM = 128
BOND_DIM = 16
DEPTH = 6

NC, NS = 2, 16
NW = NC * NS
EW = N_EDGES // NW
CH = 200
HALF = CH // 2
NCH = EW // CH
G3 = N_EDGES // CH
NPT = 624
NTAIL = N_NODES - NPT * NS

def _sc_mesh():
    return plsc.VectorSubcoreMesh(
        core_axis_name="c", subcore_axis_name="s", num_cores=NC, num_subcores=NS)


def _worker_id():
    return lax.axis_index("s") * NC + lax.axis_index("c")


NT = NCH // 2


def _sc_gather(table, idx3):
    nrow = table.shape[0]

    dt = table.dtype

    def body(table_ref, idx3_ref, out_ref, idxbuf, rowbuf, gs0, gs1, os0, os1):
        wid = _worker_id()
        pltpu.sync_copy(idx3_ref.at[pl.ds(wid * NCH, NCH)], idxbuf)

        def issue_gather(ci, b, sem):
            pltpu.async_copy(table_ref.at[idxbuf.at[ci, 0]],
                             rowbuf.at[b, pl.ds(0, HALF)], sem)
            pltpu.async_copy(table_ref.at[idxbuf.at[ci, 1]],
                             rowbuf.at[b, pl.ds(HALF, HALF)], sem)

        def wait_gather(b, sem):
            pltpu.make_async_copy(table_ref.at[pl.ds(0, CH)],
                                  rowbuf.at[b], sem).wait()

        def wait_store(b, sem):
            pltpu.make_async_copy(rowbuf.at[b],
                                  out_ref.at[pl.ds(0, CH)], sem).wait()

        issue_gather(0, 0, gs0)
        issue_gather(1, 1, gs1)

        def step(t, carry):
            c0 = 2 * t
            wait_gather(0, gs0)
            pltpu.async_copy(rowbuf.at[0],
                             out_ref.at[pl.ds((wid * NCH + c0) * CH, CH)], os0)
            wait_gather(1, gs1)
            pltpu.async_copy(rowbuf.at[1],
                             out_ref.at[pl.ds((wid * NCH + c0 + 1) * CH, CH)],
                             os1)

            @pl.when(t + 1 < NT)
            def _next():
                wait_store(0, os0)
                issue_gather(c0 + 2, 0, gs0)
                wait_store(1, os1)
                issue_gather(c0 + 3, 1, gs1)

            return carry

        lax.fori_loop(0, NT, step, 0)
        wait_store(0, os0)
        wait_store(1, os1)

    f = pl.kernel(
        body,
        out_type=jax.ShapeDtypeStruct((N_EDGES, OUT_DIM), dt),
        mesh=_sc_mesh(),
        scratch_types=[
            pltpu.VMEM((NCH, 2, HALF), jnp.int32),
            pltpu.VMEM((2, CH, OUT_DIM), dt),
            pltpu.SemaphoreType.DMA,
            pltpu.SemaphoreType.DMA,
            pltpu.SemaphoreType.DMA,
            pltpu.SemaphoreType.DMA,
        ],
        name=f"sc_gather_{nrow}",
    )
    return f(table, idx3)


CHS = 80
NCHS = EW // CHS
NTS = (NCHS + 1) // 2


def _sc_scatter(rows, idxw, zeros):
    def body(rows_ref, idxw_ref, zeros_ref, parts_ref, idxbuf, rowbuf, esum,
             ls0, ls1):
        cid = lax.axis_index("c")
        sid = lax.axis_index("s")
        wid = _worker_id()

        pltpu.sync_copy(idxw_ref.at[wid], idxbuf)

        def issue_load(ci, b, sem):
            pltpu.async_copy(rows_ref.at[pl.ds((wid * NCHS + ci) * CHS, CHS)],
                             rowbuf.at[b], sem)

        def wait_load(b, sem):
            pltpu.make_async_copy(rows_ref.at[pl.ds(0, CHS)],
                                  rowbuf.at[b], sem).wait()

        def scatter(ci, b):
            pltpu.sync_copy(rowbuf.at[b], esum.at[idxbuf.at[ci]], add=True)

        issue_load(0, 0, ls0)
        issue_load(1, 1, ls1)

        pltpu.sync_copy(zeros_ref.at[pl.ds(sid * NPT, NPT)],
                        esum.at[pl.ds(sid * NPT, NPT)])

        @pl.when(sid == 0)
        def _init_tail():
            pltpu.sync_copy(zeros_ref.at[pl.ds(NPT * NS, NTAIL)],
                            esum.at[pl.ds(NPT * NS, NTAIL)])

        plsc.subcore_barrier()

        def step(t, carry):
            c0 = 2 * t
            wait_load(0, ls0)
            scatter(c0, 0)

            @pl.when(c0 + 2 < NCHS)
            def _next0():
                issue_load(c0 + 2, 0, ls0)

            @pl.when(c0 + 1 < NCHS)
            def _odd():
                wait_load(1, ls1)
                scatter(c0 + 1, 1)

                @pl.when(c0 + 3 < NCHS)
                def _next1():
                    issue_load(c0 + 3, 1, ls1)

            return carry

        lax.fori_loop(0, NTS, step, 0)
        plsc.subcore_barrier()
        pltpu.sync_copy(esum.at[pl.ds(sid * NPT, NPT)],
                        parts_ref.at[cid, pl.ds(sid * NPT, NPT)])

        @pl.when(sid == 0)
        def _out_tail():
            pltpu.sync_copy(esum.at[pl.ds(NPT * NS, NTAIL)],
                            parts_ref.at[cid, pl.ds(NPT * NS, NTAIL)])

    f = pl.kernel(
        body,
        out_type=jax.ShapeDtypeStruct((NC, N_NODES, OUT_DIM), jnp.float32),
        mesh=_sc_mesh(),
        scratch_types=[
            pltpu.VMEM((NCHS, CHS), jnp.int32),
            pltpu.VMEM((2, CHS, OUT_DIM), jnp.float32),
            pltpu.VMEM_SHARED((N_NODES, OUT_DIM), jnp.float32),
            pltpu.SemaphoreType.DMA,
            pltpu.SemaphoreType.DMA,
        ],
        name="sc_scatter_add",
    )
    return f(rows, idxw, zeros)


_EB = 1280
_EG = N_EDGES // _EB
_NB = 1000
_NG = N_NODES // _NB


def _p0_body(xg_ref, ea_ref, wx_ref, wb_ref, inp_ref, msg_ref):
    v = jnp.dot(xg_ref[...], wx_ref[...], preferred_element_type=jnp.float32)
    v += jnp.dot(ea_ref[...], wb_ref[...], preferred_element_type=jnp.float32)
    inp_ref[...] = v.astype(inp_ref.dtype)
    msg_ref[...] = jnp.maximum(v, 0.0)


def _tc_p0(xg, ea, wxt, wbt):
    return pl.pallas_call(
        _p0_body,
        grid=(_EG,),
        in_specs=[
            pl.BlockSpec((_EB, ATOM_DIM), lambda i: (i, 0)),
            pl.BlockSpec((_EB, BOND_DIM), lambda i: (i, 0)),
            pl.BlockSpec((ATOM_DIM, OUT_DIM), lambda i: (0, 0)),
            pl.BlockSpec((BOND_DIM, OUT_DIM), lambda i: (0, 0)),
        ],
        out_specs=[
            pl.BlockSpec((_EB, OUT_DIM), lambda i: (i, 0)),
            pl.BlockSpec((_EB, OUT_DIM), lambda i: (i, 0)),
        ],
        out_shape=[
            jax.ShapeDtypeStruct((N_EDGES, OUT_DIM), jnp.bfloat16),
            jax.ShapeDtypeStruct((N_EDGES, OUT_DIM), jnp.float32),
        ],
    )(xg, ea, wxt, wbt)


def _swap_pairs(a3):
    return jnp.concatenate([a3[:, 1:2, :], a3[:, 0:1, :]], axis=1)


def _depth_body(inp_ref, msg_ref, gm_ref, wh_ref, out_ref):
    t = jnp.dot(msg_ref[...], wh_ref[...], preferred_element_type=jnp.float32)
    d3 = (gm_ref[...] - t).reshape(_EB // 2, 2, OUT_DIM)
    sw = _swap_pairs(d3).reshape(_EB, OUT_DIM)
    out_ref[...] = jnp.maximum(inp_ref[...].astype(jnp.float32) + sw, 0.0)


def _tc_depth(inp, msg, gm, wht):
    return pl.pallas_call(
        _depth_body,
        grid=(_EG,),
        in_specs=[
            pl.BlockSpec((_EB, OUT_DIM), lambda i: (i, 0)),
            pl.BlockSpec((_EB, OUT_DIM), lambda i: (i, 0)),
            pl.BlockSpec((_EB, OUT_DIM), lambda i: (i, 0)),
            pl.BlockSpec((OUT_DIM, OUT_DIM), lambda i: (0, 0)),
        ],
        out_specs=pl.BlockSpec((_EB, OUT_DIM), lambda i: (i, 0)),
        out_shape=jax.ShapeDtypeStruct((N_EDGES, OUT_DIM), jnp.float32),
    )(inp, msg, gm, wht)


def _g_body(parts_ref, wh_ref, out_ref):
    p = parts_ref[...]
    out_ref[...] = jnp.dot(p[0] + p[1], wh_ref[...],
                           preferred_element_type=jnp.float32)


def _tc_g(parts, wht):
    return pl.pallas_call(
        _g_body,
        grid=(_NG,),
        in_specs=[
            pl.BlockSpec((NC, _NB, OUT_DIM), lambda i: (0, i, 0)),
            pl.BlockSpec((OUT_DIM, OUT_DIM), lambda i: (0, 0)),
        ],
        out_specs=pl.BlockSpec((_NB, OUT_DIM), lambda i: (i, 0)),
        out_shape=jax.ShapeDtypeStruct((N_NODES, OUT_DIM), jnp.float32),
    )(parts, wht)


def _final_body(x_ref, parts_ref, wox_ref, woo_ref, bo_ref, out_ref):
    p = parts_ref[...]
    h = jnp.dot(x_ref[...], wox_ref[...], preferred_element_type=jnp.float32)
    h += jnp.dot(p[0] + p[1], woo_ref[...], preferred_element_type=jnp.float32)
    out_ref[...] = jnp.maximum(h + bo_ref[...], 0.0)


def _tc_final(x, parts, woxt, woot, bo):
    return pl.pallas_call(
        _final_body,
        grid=(_NG,),
        in_specs=[
            pl.BlockSpec((_NB, ATOM_DIM), lambda i: (i, 0)),
            pl.BlockSpec((NC, _NB, OUT_DIM), lambda i: (0, i, 0)),
            pl.BlockSpec((ATOM_DIM, OUT_DIM), lambda i: (0, 0)),
            pl.BlockSpec((OUT_DIM, OUT_DIM), lambda i: (0, 0)),
            pl.BlockSpec((1, OUT_DIM), lambda i: (0, 0)),
        ],
        out_specs=pl.BlockSpec((_NB, OUT_DIM), lambda i: (i, 0)),
        out_shape=jax.ShapeDtypeStruct((N_NODES, OUT_DIM), jnp.float32),
    )(x, parts, woxt, woot, bo)


def kernel(x, edge_index, edge_attr, W_i, W_h, W_o, b_o):
    src = edge_index[0].astype(jnp.int32)
    dst = edge_index[1].astype(jnp.int32)
    src3 = src.reshape(G3, 2, HALF)
    dst3 = dst.reshape(G3, 2, HALF)
    dstw = dst.reshape(NW, NCHS, CHS)
    zeros = jnp.zeros((N_NODES, OUT_DIM), jnp.float32)

    wxt = W_i[:, :ATOM_DIM].T
    wbt = W_i[:, ATOM_DIM:].T
    wht = W_h.T
    woxt = W_o[:, :ATOM_DIM].T
    woot = W_o[:, ATOM_DIM:].T
    bo2 = b_o.reshape(1, OUT_DIM)

    xg = _sc_gather(x, src3)
    inp, msg = _tc_p0(xg, edge_attr, wxt, wbt)

    parts = _sc_scatter(msg, dstw, zeros)
    for _ in range(DEPTH - 1):
        g = _tc_g(parts, wht)
        gm = _sc_gather(g, dst3)
        msg = _tc_depth(inp, msg, gm, wht)
        parts = _sc_scatter(msg, dstw, zeros)

    return _tc_final(x, parts, woxt, woot, bo2)


# speedup vs baseline: 1.6919x; 1.1052x over previous
import jax
import jax.numpy as jnp
from jax import lax
from jax.experimental import pallas as pl
from jax.experimental.pallas import tpu as pltpu
from jax.experimental.pallas import tpu_sc as plsc

N_NODES = 10000
N_EDGES = 320000
ATOM_DIM = 128
OUT_DIM = 128
BOND_DIM = 16
DEPTH = 6

NC, NS = 2, 16
NW = NC * NS
EW = N_EDGES // NW
CH = 200
HALF = CH // 2
NCH = EW // CH
G3 = N_EDGES // CH
NPT = 624
NTAIL = N_NODES - NPT * NS

def _sc_mesh():
    return plsc.VectorSubcoreMesh(
        core_axis_name="c", subcore_axis_name="s", num_cores=NC, num_subcores=NS)


def _worker_id():
    return lax.axis_index("s") * NC + lax.axis_index("c")


NT = NCH // 2


def _sc_gather(table, idx3):
    nrow = table.shape[0]

    dt = table.dtype

    def body(table_ref, idx3_ref, out_ref, idxbuf, rowbuf, gs0, gs1, os0, os1):
        wid = _worker_id()
        pltpu.sync_copy(idx3_ref.at[pl.ds(wid * NCH, NCH)], idxbuf)

        def issue_gather(ci, b, sem):
            pltpu.async_copy(table_ref.at[idxbuf.at[ci, 0]],
                             rowbuf.at[b, pl.ds(0, HALF)], sem)
            pltpu.async_copy(table_ref.at[idxbuf.at[ci, 1]],
                             rowbuf.at[b, pl.ds(HALF, HALF)], sem)

        def wait_gather(b, sem):
            pltpu.make_async_copy(table_ref.at[pl.ds(0, CH)],
                                  rowbuf.at[b], sem).wait()

        def wait_store(b, sem):
            pltpu.make_async_copy(rowbuf.at[b],
                                  out_ref.at[pl.ds(0, CH)], sem).wait()

        issue_gather(0, 0, gs0)
        issue_gather(1, 1, gs1)

        def step(t, carry):
            c0 = 2 * t
            wait_gather(0, gs0)
            pltpu.async_copy(rowbuf.at[0],
                             out_ref.at[pl.ds((wid * NCH + c0) * CH, CH)], os0)
            wait_gather(1, gs1)
            pltpu.async_copy(rowbuf.at[1],
                             out_ref.at[pl.ds((wid * NCH + c0 + 1) * CH, CH)],
                             os1)

            @pl.when(t + 1 < NT)
            def _next():
                wait_store(0, os0)
                issue_gather(c0 + 2, 0, gs0)
                wait_store(1, os1)
                issue_gather(c0 + 3, 1, gs1)

            return carry

        lax.fori_loop(0, NT, step, 0)
        wait_store(0, os0)
        wait_store(1, os1)

    f = pl.kernel(
        body,
        out_type=jax.ShapeDtypeStruct((N_EDGES, OUT_DIM), dt),
        mesh=_sc_mesh(),
        scratch_types=[
            pltpu.VMEM((NCH, 2, HALF), jnp.int32),
            pltpu.VMEM((2, CH, OUT_DIM), dt),
            pltpu.SemaphoreType.DMA,
            pltpu.SemaphoreType.DMA,
            pltpu.SemaphoreType.DMA,
            pltpu.SemaphoreType.DMA,
        ],
        name=f"sc_gather_{nrow}",
    )
    return f(table, idx3)


CHS = 80
NCHS = EW // CHS
NTS = (NCHS + 1) // 2


def _sc_scatter(rows, idxw, zeros):
    def body(rows_ref, idxw_ref, zeros_ref, parts_ref, idxbuf, rowbuf, esum,
             ls0, ls1):
        cid = lax.axis_index("c")
        sid = lax.axis_index("s")
        wid = _worker_id()

        pltpu.sync_copy(idxw_ref.at[wid], idxbuf)

        def issue_load(ci, b, sem):
            pltpu.async_copy(rows_ref.at[pl.ds((wid * NCHS + ci) * CHS, CHS)],
                             rowbuf.at[b], sem)

        def wait_load(b, sem):
            pltpu.make_async_copy(rows_ref.at[pl.ds(0, CHS)],
                                  rowbuf.at[b], sem).wait()

        def scatter(ci, b):
            pltpu.sync_copy(rowbuf.at[b], esum.at[idxbuf.at[ci]], add=True)

        issue_load(0, 0, ls0)
        issue_load(1, 1, ls1)

        pltpu.sync_copy(zeros_ref.at[pl.ds(sid * NPT, NPT)],
                        esum.at[pl.ds(sid * NPT, NPT)])

        @pl.when(sid == 0)
        def _init_tail():
            pltpu.sync_copy(zeros_ref.at[pl.ds(NPT * NS, NTAIL)],
                            esum.at[pl.ds(NPT * NS, NTAIL)])

        plsc.subcore_barrier()

        def step(t, carry):
            c0 = 2 * t
            wait_load(0, ls0)
            scatter(c0, 0)

            @pl.when(c0 + 2 < NCHS)
            def _next0():
                issue_load(c0 + 2, 0, ls0)

            @pl.when(c0 + 1 < NCHS)
            def _odd():
                wait_load(1, ls1)
                scatter(c0 + 1, 1)

                @pl.when(c0 + 3 < NCHS)
                def _next1():
                    issue_load(c0 + 3, 1, ls1)

            return carry

        lax.fori_loop(0, NTS, step, 0)
        plsc.subcore_barrier()
        pltpu.sync_copy(esum.at[pl.ds(sid * NPT, NPT)],
                        parts_ref.at[cid, pl.ds(sid * NPT, NPT)])

        @pl.when(sid == 0)
        def _out_tail():
            pltpu.sync_copy(esum.at[pl.ds(NPT * NS, NTAIL)],
                            parts_ref.at[cid, pl.ds(NPT * NS, NTAIL)])

    f = pl.kernel(
        body,
        out_type=jax.ShapeDtypeStruct((NC, N_NODES, OUT_DIM), jnp.float32),
        mesh=_sc_mesh(),
        scratch_types=[
            pltpu.VMEM((NCHS, CHS), jnp.int32),
            pltpu.VMEM((2, CHS, OUT_DIM), jnp.float32),
            pltpu.VMEM_SHARED((N_NODES, OUT_DIM), jnp.float32),
            pltpu.SemaphoreType.DMA,
            pltpu.SemaphoreType.DMA,
        ],
        name="sc_scatter_add",
    )
    return f(rows, idxw, zeros)


_EB = 2000
_EG = N_EDGES // _EB
_NB = 1000
_NG = N_NODES // _NB


def _p0_body(xg_ref, ea_ref, wx_ref, wb_ref, inp_ref, msg_ref):
    v = jnp.dot(xg_ref[...], wx_ref[...], preferred_element_type=jnp.float32)
    v += jnp.dot(ea_ref[...], wb_ref[...], preferred_element_type=jnp.float32)
    inp_ref[...] = v.astype(inp_ref.dtype)
    msg_ref[...] = jnp.maximum(v, 0.0)


def _tc_p0(xg, ea, wxt, wbt):
    return pl.pallas_call(
        _p0_body,
        grid=(_EG,),
        in_specs=[
            pl.BlockSpec((_EB, ATOM_DIM), lambda i: (i, 0)),
            pl.BlockSpec((_EB, BOND_DIM), lambda i: (i, 0)),
            pl.BlockSpec((ATOM_DIM, OUT_DIM), lambda i: (0, 0)),
            pl.BlockSpec((BOND_DIM, OUT_DIM), lambda i: (0, 0)),
        ],
        out_specs=[
            pl.BlockSpec((_EB, OUT_DIM), lambda i: (i, 0)),
            pl.BlockSpec((_EB, OUT_DIM), lambda i: (i, 0)),
        ],
        out_shape=[
            jax.ShapeDtypeStruct((N_EDGES, OUT_DIM), jnp.bfloat16),
            jax.ShapeDtypeStruct((N_EDGES, OUT_DIM), jnp.float32),
        ],
    )(xg, ea, wxt, wbt)


def _swap_pairs(a3):
    return jnp.concatenate([a3[:, 1:2, :], a3[:, 0:1, :]], axis=1)


def _depth_body(inp_ref, msg_ref, gm_ref, wh_ref, out_ref):
    t = jnp.dot(msg_ref[...], wh_ref[...], preferred_element_type=jnp.float32)
    d3 = (gm_ref[...] - t).reshape(_EB // 2, 2, OUT_DIM)
    sw = _swap_pairs(d3).reshape(_EB, OUT_DIM)
    out_ref[...] = jnp.maximum(inp_ref[...].astype(jnp.float32) + sw, 0.0)


def _tc_depth(inp, msg, gm, wht):
    return pl.pallas_call(
        _depth_body,
        grid=(_EG,),
        in_specs=[
            pl.BlockSpec((_EB, OUT_DIM), lambda i: (i, 0)),
            pl.BlockSpec((_EB, OUT_DIM), lambda i: (i, 0)),
            pl.BlockSpec((_EB, OUT_DIM), lambda i: (i, 0)),
            pl.BlockSpec((OUT_DIM, OUT_DIM), lambda i: (0, 0)),
        ],
        out_specs=pl.BlockSpec((_EB, OUT_DIM), lambda i: (i, 0)),
        out_shape=jax.ShapeDtypeStruct((N_EDGES, OUT_DIM), jnp.float32),
    )(inp, msg, gm, wht)


def _g_body(parts_ref, wh_ref, out_ref):
    p = parts_ref[...]
    out_ref[...] = jnp.dot(p[0] + p[1], wh_ref[...],
                           preferred_element_type=jnp.float32)


def _tc_g(parts, wht):
    return pl.pallas_call(
        _g_body,
        grid=(_NG,),
        in_specs=[
            pl.BlockSpec((NC, _NB, OUT_DIM), lambda i: (0, i, 0)),
            pl.BlockSpec((OUT_DIM, OUT_DIM), lambda i: (0, 0)),
        ],
        out_specs=pl.BlockSpec((_NB, OUT_DIM), lambda i: (i, 0)),
        out_shape=jax.ShapeDtypeStruct((N_NODES, OUT_DIM), jnp.float32),
    )(parts, wht)


def _final_body(x_ref, parts_ref, wox_ref, woo_ref, bo_ref, out_ref):
    p = parts_ref[...]
    h = jnp.dot(x_ref[...], wox_ref[...], preferred_element_type=jnp.float32)
    h += jnp.dot(p[0] + p[1], woo_ref[...], preferred_element_type=jnp.float32)
    out_ref[...] = jnp.maximum(h + bo_ref[...], 0.0)


def _tc_final(x, parts, woxt, woot, bo):
    return pl.pallas_call(
        _final_body,
        grid=(_NG,),
        in_specs=[
            pl.BlockSpec((_NB, ATOM_DIM), lambda i: (i, 0)),
            pl.BlockSpec((NC, _NB, OUT_DIM), lambda i: (0, i, 0)),
            pl.BlockSpec((ATOM_DIM, OUT_DIM), lambda i: (0, 0)),
            pl.BlockSpec((OUT_DIM, OUT_DIM), lambda i: (0, 0)),
            pl.BlockSpec((1, OUT_DIM), lambda i: (0, 0)),
        ],
        out_specs=pl.BlockSpec((_NB, OUT_DIM), lambda i: (i, 0)),
        out_shape=jax.ShapeDtypeStruct((N_NODES, OUT_DIM), jnp.float32),
    )(x, parts, woxt, woot, bo)


def kernel(x, edge_index, edge_attr, W_i, W_h, W_o, b_o):
    src = edge_index[0].astype(jnp.int32)
    dst = edge_index[1].astype(jnp.int32)
    src3 = src.reshape(G3, 2, HALF)
    dst3 = dst.reshape(G3, 2, HALF)
    dstw = dst.reshape(NW, NCHS, CHS)
    zeros = jnp.zeros((N_NODES, OUT_DIM), jnp.float32)

    wxt = W_i[:, :ATOM_DIM].T
    wbt = W_i[:, ATOM_DIM:].T
    wht = W_h.T
    woxt = W_o[:, :ATOM_DIM].T
    woot = W_o[:, ATOM_DIM:].T
    bo2 = b_o.reshape(1, OUT_DIM)

    xg = _sc_gather(x, src3)
    inp, msg = _tc_p0(xg, edge_attr, wxt, wbt)

    parts = _sc_scatter(msg, dstw, zeros)
    for _ in range(DEPTH - 1):
        g = _tc_g(parts, wht)
        gm = _sc_gather(g, dst3)
        msg = _tc_depth(inp, msg, gm, wht)
        parts = _sc_scatter(msg, dstw, zeros)

    return _tc_final(x, parts, woxt, woot, bo2)


# speedup vs baseline: 1.8728x; 1.1069x over previous
import jax
import jax.numpy as jnp
from jax import lax
from jax.experimental import pallas as pl
from jax.experimental.pallas import tpu as pltpu
from jax.experimental.pallas import tpu_sc as plsc

N_NODES = 10000
N_EDGES = 320000
ATOM_DIM = 128
OUT_DIM = 128
BOND_DIM = 16
DEPTH = 6

NC, NS = 2, 16
NW = NC * NS
EW = N_EDGES // NW
CH = 200
HALF = CH // 2
NCH = EW // CH
G3 = N_EDGES // CH
NPT = 624
NTAIL = N_NODES - NPT * NS

def _sc_mesh():
    return plsc.VectorSubcoreMesh(
        core_axis_name="c", subcore_axis_name="s", num_cores=NC, num_subcores=NS)


def _worker_id():
    return lax.axis_index("s") * NC + lax.axis_index("c")


NT = NCH // 2


def _sc_gather(table, idx3):
    nrow = table.shape[0]

    dt = table.dtype

    def body(table_ref, idx3_ref, out_ref, idxbuf, rowbuf, gs0, gs1, os0, os1):
        wid = _worker_id()
        pltpu.sync_copy(idx3_ref.at[pl.ds(wid * NCH, NCH)], idxbuf)

        def issue_gather(ci, b, sem):
            pltpu.async_copy(table_ref.at[idxbuf.at[ci, 0]],
                             rowbuf.at[b, pl.ds(0, HALF)], sem)
            pltpu.async_copy(table_ref.at[idxbuf.at[ci, 1]],
                             rowbuf.at[b, pl.ds(HALF, HALF)], sem)

        def wait_gather(b, sem):
            pltpu.make_async_copy(table_ref.at[pl.ds(0, CH)],
                                  rowbuf.at[b], sem).wait()

        def wait_store(b, sem):
            pltpu.make_async_copy(rowbuf.at[b],
                                  out_ref.at[pl.ds(0, CH)], sem).wait()

        issue_gather(0, 0, gs0)
        issue_gather(1, 1, gs1)

        def step(t, carry):
            c0 = 2 * t
            wait_gather(0, gs0)
            pltpu.async_copy(rowbuf.at[0],
                             out_ref.at[pl.ds((wid * NCH + c0) * CH, CH)], os0)
            wait_gather(1, gs1)
            pltpu.async_copy(rowbuf.at[1],
                             out_ref.at[pl.ds((wid * NCH + c0 + 1) * CH, CH)],
                             os1)

            @pl.when(t + 1 < NT)
            def _next():
                wait_store(0, os0)
                issue_gather(c0 + 2, 0, gs0)
                wait_store(1, os1)
                issue_gather(c0 + 3, 1, gs1)

            return carry

        lax.fori_loop(0, NT, step, 0)
        wait_store(0, os0)
        wait_store(1, os1)

    f = pl.kernel(
        body,
        out_type=jax.ShapeDtypeStruct((N_EDGES, OUT_DIM), dt),
        mesh=_sc_mesh(),
        scratch_types=[
            pltpu.VMEM((NCH, 2, HALF), jnp.int32),
            pltpu.VMEM((2, CH, OUT_DIM), dt),
            pltpu.SemaphoreType.DMA,
            pltpu.SemaphoreType.DMA,
            pltpu.SemaphoreType.DMA,
            pltpu.SemaphoreType.DMA,
        ],
        name=f"sc_gather_{nrow}",
    )
    return f(table, idx3)


CHS = 80
NCHS = EW // CHS
NTS = (NCHS + 1) // 2


def _sc_scatter(rows, idxw, zeros):
    def body(rows_ref, idxw_ref, zeros_ref, parts_ref, idxbuf, rowbuf, esum,
             ls0, ls1):
        cid = lax.axis_index("c")
        sid = lax.axis_index("s")
        wid = _worker_id()

        pltpu.sync_copy(idxw_ref.at[wid], idxbuf)

        def issue_load(ci, b, sem):
            pltpu.async_copy(rows_ref.at[pl.ds((wid * NCHS + ci) * CHS, CHS)],
                             rowbuf.at[b], sem)

        def wait_load(b, sem):
            pltpu.make_async_copy(rows_ref.at[pl.ds(0, CHS)],
                                  rowbuf.at[b], sem).wait()

        def scatter(ci, b):
            pltpu.sync_copy(rowbuf.at[b], esum.at[idxbuf.at[ci]], add=True)

        issue_load(0, 0, ls0)
        issue_load(1, 1, ls1)

        pltpu.sync_copy(zeros_ref.at[pl.ds(sid * NPT, NPT)],
                        esum.at[pl.ds(sid * NPT, NPT)])

        @pl.when(sid == 0)
        def _init_tail():
            pltpu.sync_copy(zeros_ref.at[pl.ds(NPT * NS, NTAIL)],
                            esum.at[pl.ds(NPT * NS, NTAIL)])

        plsc.subcore_barrier()

        def step(t, carry):
            c0 = 2 * t
            wait_load(0, ls0)
            scatter(c0, 0)

            @pl.when(c0 + 2 < NCHS)
            def _next0():
                issue_load(c0 + 2, 0, ls0)

            @pl.when(c0 + 1 < NCHS)
            def _odd():
                wait_load(1, ls1)
                scatter(c0 + 1, 1)

                @pl.when(c0 + 3 < NCHS)
                def _next1():
                    issue_load(c0 + 3, 1, ls1)

            return carry

        lax.fori_loop(0, NTS, step, 0)
        plsc.subcore_barrier()
        pltpu.sync_copy(esum.at[pl.ds(sid * NPT, NPT)],
                        parts_ref.at[cid, pl.ds(sid * NPT, NPT)])

        @pl.when(sid == 0)
        def _out_tail():
            pltpu.sync_copy(esum.at[pl.ds(NPT * NS, NTAIL)],
                            parts_ref.at[cid, pl.ds(NPT * NS, NTAIL)])

    f = pl.kernel(
        body,
        out_type=jax.ShapeDtypeStruct((NC, N_NODES, OUT_DIM), jnp.float32),
        mesh=_sc_mesh(),
        scratch_types=[
            pltpu.VMEM((NCHS, CHS), jnp.int32),
            pltpu.VMEM((2, CHS, OUT_DIM), jnp.float32),
            pltpu.VMEM_SHARED((N_NODES, OUT_DIM), jnp.float32),
            pltpu.SemaphoreType.DMA,
            pltpu.SemaphoreType.DMA,
        ],
        name="sc_scatter_add",
    )
    return f(rows, idxw, zeros)


_EB = 4000
_EG = N_EDGES // _EB
_NB = 1000
_NG = N_NODES // _NB


def _p0_body(xg_ref, ea_ref, wx_ref, wb_ref, inp_ref, msg_ref):
    v = jnp.dot(xg_ref[...], wx_ref[...], preferred_element_type=jnp.float32)
    v += jnp.dot(ea_ref[...], wb_ref[...], preferred_element_type=jnp.float32)
    inp_ref[...] = v.astype(inp_ref.dtype)
    msg_ref[...] = jnp.maximum(v, 0.0)


def _tc_p0(xg, ea, wxt, wbt):
    return pl.pallas_call(
        _p0_body,
        grid=(_EG,),
        in_specs=[
            pl.BlockSpec((_EB, ATOM_DIM), lambda i: (i, 0)),
            pl.BlockSpec((_EB, BOND_DIM), lambda i: (i, 0)),
            pl.BlockSpec((ATOM_DIM, OUT_DIM), lambda i: (0, 0)),
            pl.BlockSpec((BOND_DIM, OUT_DIM), lambda i: (0, 0)),
        ],
        out_specs=[
            pl.BlockSpec((_EB, OUT_DIM), lambda i: (i, 0)),
            pl.BlockSpec((_EB, OUT_DIM), lambda i: (i, 0)),
        ],
        out_shape=[
            jax.ShapeDtypeStruct((N_EDGES, OUT_DIM), jnp.bfloat16),
            jax.ShapeDtypeStruct((N_EDGES, OUT_DIM), jnp.float32),
        ],
    )(xg, ea, wxt, wbt)


def _swap_pairs(a3):
    return jnp.concatenate([a3[:, 1:2, :], a3[:, 0:1, :]], axis=1)


def _depth_body(inp_ref, msg_ref, gm_ref, wh_ref, out_ref):
    t = jnp.dot(msg_ref[...], wh_ref[...], preferred_element_type=jnp.float32)
    d3 = (gm_ref[...] - t).reshape(_EB // 2, 2, OUT_DIM)
    sw = _swap_pairs(d3).reshape(_EB, OUT_DIM)
    out_ref[...] = jnp.maximum(inp_ref[...].astype(jnp.float32) + sw, 0.0)


def _tc_depth(inp, msg, gm, wht):
    return pl.pallas_call(
        _depth_body,
        grid=(_EG,),
        in_specs=[
            pl.BlockSpec((_EB, OUT_DIM), lambda i: (i, 0)),
            pl.BlockSpec((_EB, OUT_DIM), lambda i: (i, 0)),
            pl.BlockSpec((_EB, OUT_DIM), lambda i: (i, 0)),
            pl.BlockSpec((OUT_DIM, OUT_DIM), lambda i: (0, 0)),
        ],
        out_specs=pl.BlockSpec((_EB, OUT_DIM), lambda i: (i, 0)),
        out_shape=jax.ShapeDtypeStruct((N_EDGES, OUT_DIM), jnp.float32),
    )(inp, msg, gm, wht)


def _g_body(parts_ref, wh_ref, out_ref):
    p = parts_ref[...]
    out_ref[...] = jnp.dot(p[0] + p[1], wh_ref[...],
                           preferred_element_type=jnp.float32)


def _tc_g(parts, wht):
    return pl.pallas_call(
        _g_body,
        grid=(_NG,),
        in_specs=[
            pl.BlockSpec((NC, _NB, OUT_DIM), lambda i: (0, i, 0)),
            pl.BlockSpec((OUT_DIM, OUT_DIM), lambda i: (0, 0)),
        ],
        out_specs=pl.BlockSpec((_NB, OUT_DIM), lambda i: (i, 0)),
        out_shape=jax.ShapeDtypeStruct((N_NODES, OUT_DIM), jnp.float32),
    )(parts, wht)


def _final_body(x_ref, parts_ref, wox_ref, woo_ref, bo_ref, out_ref):
    p = parts_ref[...]
    h = jnp.dot(x_ref[...], wox_ref[...], preferred_element_type=jnp.float32)
    h += jnp.dot(p[0] + p[1], woo_ref[...], preferred_element_type=jnp.float32)
    out_ref[...] = jnp.maximum(h + bo_ref[...], 0.0)


def _tc_final(x, parts, woxt, woot, bo):
    return pl.pallas_call(
        _final_body,
        grid=(_NG,),
        in_specs=[
            pl.BlockSpec((_NB, ATOM_DIM), lambda i: (i, 0)),
            pl.BlockSpec((NC, _NB, OUT_DIM), lambda i: (0, i, 0)),
            pl.BlockSpec((ATOM_DIM, OUT_DIM), lambda i: (0, 0)),
            pl.BlockSpec((OUT_DIM, OUT_DIM), lambda i: (0, 0)),
            pl.BlockSpec((1, OUT_DIM), lambda i: (0, 0)),
        ],
        out_specs=pl.BlockSpec((_NB, OUT_DIM), lambda i: (i, 0)),
        out_shape=jax.ShapeDtypeStruct((N_NODES, OUT_DIM), jnp.float32),
    )(x, parts, woxt, woot, bo)


def kernel(x, edge_index, edge_attr, W_i, W_h, W_o, b_o):
    src = edge_index[0].astype(jnp.int32)
    dst = edge_index[1].astype(jnp.int32)
    src3 = src.reshape(G3, 2, HALF)
    dst3 = dst.reshape(G3, 2, HALF)
    dstw = dst.reshape(NW, NCHS, CHS)
    zeros = jnp.zeros((N_NODES, OUT_DIM), jnp.float32)

    wxt = W_i[:, :ATOM_DIM].T
    wbt = W_i[:, ATOM_DIM:].T
    wht = W_h.T
    woxt = W_o[:, :ATOM_DIM].T
    woot = W_o[:, ATOM_DIM:].T
    bo2 = b_o.reshape(1, OUT_DIM)

    xg = _sc_gather(x, src3)
    inp, msg = _tc_p0(xg, edge_attr, wxt, wbt)

    parts = _sc_scatter(msg, dstw, zeros)
    for _ in range(DEPTH - 1):
        g = _tc_g(parts, wht)
        gm = _sc_gather(g, dst3)
        msg = _tc_depth(inp, msg, gm, wht)
        parts = _sc_scatter(msg, dstw, zeros)

    return _tc_final(x, parts, woxt, woot, bo2)


# speedup vs baseline: 1.9443x; 1.0382x over previous
import jax
import jax.numpy as jnp
from jax import lax
from jax.experimental import pallas as pl
from jax.experimental.pallas import tpu as pltpu
from jax.experimental.pallas import tpu_sc as plsc

N_NODES = 10000
N_EDGES = 320000
ATOM_DIM = 128
OUT_DIM = 128
BOND_DIM = 16
DEPTH = 6

NC, NS = 2, 16
NW = NC * NS
EW = N_EDGES // NW
CH = 200
HALF = CH // 2
NCH = EW // CH
G3 = N_EDGES // CH
NPT = 624
NTAIL = N_NODES - NPT * NS

def _sc_mesh():
    return plsc.VectorSubcoreMesh(
        core_axis_name="c", subcore_axis_name="s", num_cores=NC, num_subcores=NS)


def _worker_id():
    return lax.axis_index("s") * NC + lax.axis_index("c")


NT = NCH // 2


def _sc_gather(table, idx3):
    nrow = table.shape[0]

    dt = table.dtype

    def body(table_ref, idx3_ref, out_ref, idxbuf, rowbuf, gs0, gs1, os0, os1):
        wid = _worker_id()
        pltpu.sync_copy(idx3_ref.at[pl.ds(wid * NCH, NCH)], idxbuf)

        def issue_gather(ci, b, sem):
            pltpu.async_copy(table_ref.at[idxbuf.at[ci, 0]],
                             rowbuf.at[b, pl.ds(0, HALF)], sem)
            pltpu.async_copy(table_ref.at[idxbuf.at[ci, 1]],
                             rowbuf.at[b, pl.ds(HALF, HALF)], sem)

        def wait_gather(b, sem):
            pltpu.make_async_copy(table_ref.at[pl.ds(0, CH)],
                                  rowbuf.at[b], sem).wait()

        def wait_store(b, sem):
            pltpu.make_async_copy(rowbuf.at[b],
                                  out_ref.at[pl.ds(0, CH)], sem).wait()

        issue_gather(0, 0, gs0)
        issue_gather(1, 1, gs1)

        def step(t, carry):
            c0 = 2 * t
            wait_gather(0, gs0)
            pltpu.async_copy(rowbuf.at[0],
                             out_ref.at[pl.ds((wid * NCH + c0) * CH, CH)], os0)
            wait_gather(1, gs1)
            pltpu.async_copy(rowbuf.at[1],
                             out_ref.at[pl.ds((wid * NCH + c0 + 1) * CH, CH)],
                             os1)

            @pl.when(t + 1 < NT)
            def _next():
                wait_store(0, os0)
                issue_gather(c0 + 2, 0, gs0)
                wait_store(1, os1)
                issue_gather(c0 + 3, 1, gs1)

            return carry

        lax.fori_loop(0, NT, step, 0)
        wait_store(0, os0)
        wait_store(1, os1)

    f = pl.kernel(
        body,
        out_type=jax.ShapeDtypeStruct((N_EDGES, OUT_DIM), dt),
        mesh=_sc_mesh(),
        scratch_types=[
            pltpu.VMEM((NCH, 2, HALF), jnp.int32),
            pltpu.VMEM((2, CH, OUT_DIM), dt),
            pltpu.SemaphoreType.DMA,
            pltpu.SemaphoreType.DMA,
            pltpu.SemaphoreType.DMA,
            pltpu.SemaphoreType.DMA,
        ],
        name=f"sc_gather_{nrow}",
    )
    return f(table, idx3)


CHS = 80
NCHS = EW // CHS
NTS = (NCHS + 1) // 2


def _sc_scatter(rows, idxw, zeros):
    def body(rows_ref, idxw_ref, zeros_ref, parts_ref, idxbuf, rowbuf, esum,
             ls0, ls1):
        cid = lax.axis_index("c")
        sid = lax.axis_index("s")
        wid = _worker_id()

        pltpu.sync_copy(idxw_ref.at[wid], idxbuf)

        def issue_load(ci, b, sem):
            pltpu.async_copy(rows_ref.at[pl.ds((wid * NCHS + ci) * CHS, CHS)],
                             rowbuf.at[b], sem)

        def wait_load(b, sem):
            pltpu.make_async_copy(rows_ref.at[pl.ds(0, CHS)],
                                  rowbuf.at[b], sem).wait()

        def scatter(ci, b):
            pltpu.sync_copy(rowbuf.at[b], esum.at[idxbuf.at[ci]], add=True)

        issue_load(0, 0, ls0)
        issue_load(1, 1, ls1)

        pltpu.sync_copy(zeros_ref.at[pl.ds(sid * NPT, NPT)],
                        esum.at[pl.ds(sid * NPT, NPT)])

        @pl.when(sid == 0)
        def _init_tail():
            pltpu.sync_copy(zeros_ref.at[pl.ds(NPT * NS, NTAIL)],
                            esum.at[pl.ds(NPT * NS, NTAIL)])

        plsc.subcore_barrier()

        def step(t, carry):
            c0 = 2 * t
            wait_load(0, ls0)
            scatter(c0, 0)

            @pl.when(c0 + 2 < NCHS)
            def _next0():
                issue_load(c0 + 2, 0, ls0)

            @pl.when(c0 + 1 < NCHS)
            def _odd():
                wait_load(1, ls1)
                scatter(c0 + 1, 1)

                @pl.when(c0 + 3 < NCHS)
                def _next1():
                    issue_load(c0 + 3, 1, ls1)

            return carry

        lax.fori_loop(0, NTS, step, 0)
        plsc.subcore_barrier()
        pltpu.sync_copy(esum.at[pl.ds(sid * NPT, NPT)],
                        parts_ref.at[cid, pl.ds(sid * NPT, NPT)])

        @pl.when(sid == 0)
        def _out_tail():
            pltpu.sync_copy(esum.at[pl.ds(NPT * NS, NTAIL)],
                            parts_ref.at[cid, pl.ds(NPT * NS, NTAIL)])

    f = pl.kernel(
        body,
        out_type=jax.ShapeDtypeStruct((NC, N_NODES, OUT_DIM), jnp.float32),
        mesh=_sc_mesh(),
        scratch_types=[
            pltpu.VMEM((NCHS, CHS), jnp.int32),
            pltpu.VMEM((2, CHS, OUT_DIM), jnp.float32),
            pltpu.VMEM_SHARED((N_NODES, OUT_DIM), jnp.float32),
            pltpu.SemaphoreType.DMA,
            pltpu.SemaphoreType.DMA,
        ],
        name="sc_scatter_add",
    )
    return f(rows, idxw, zeros)


_EB = 8000
_EG = N_EDGES // _EB
_NB = 1000
_NG = N_NODES // _NB


def _p0_body(xg_ref, ea_ref, wx_ref, wb_ref, inp_ref, msg_ref):
    v = jnp.dot(xg_ref[...], wx_ref[...], preferred_element_type=jnp.float32)
    v += jnp.dot(ea_ref[...], wb_ref[...], preferred_element_type=jnp.float32)
    inp_ref[...] = v.astype(inp_ref.dtype)
    msg_ref[...] = jnp.maximum(v, 0.0)


def _tc_p0(xg, ea, wxt, wbt):
    return pl.pallas_call(
        _p0_body,
        grid=(_EG,),
        in_specs=[
            pl.BlockSpec((_EB, ATOM_DIM), lambda i: (i, 0)),
            pl.BlockSpec((_EB, BOND_DIM), lambda i: (i, 0)),
            pl.BlockSpec((ATOM_DIM, OUT_DIM), lambda i: (0, 0)),
            pl.BlockSpec((BOND_DIM, OUT_DIM), lambda i: (0, 0)),
        ],
        out_specs=[
            pl.BlockSpec((_EB, OUT_DIM), lambda i: (i, 0)),
            pl.BlockSpec((_EB, OUT_DIM), lambda i: (i, 0)),
        ],
        out_shape=[
            jax.ShapeDtypeStruct((N_EDGES, OUT_DIM), jnp.bfloat16),
            jax.ShapeDtypeStruct((N_EDGES, OUT_DIM), jnp.float32),
        ],
    )(xg, ea, wxt, wbt)


def _swap_pairs(a3):
    return jnp.concatenate([a3[:, 1:2, :], a3[:, 0:1, :]], axis=1)


def _depth_body(inp_ref, msg_ref, gm_ref, wh_ref, out_ref):
    t = jnp.dot(msg_ref[...], wh_ref[...], preferred_element_type=jnp.float32)
    d3 = (gm_ref[...] - t).reshape(_EB // 2, 2, OUT_DIM)
    sw = _swap_pairs(d3).reshape(_EB, OUT_DIM)
    out_ref[...] = jnp.maximum(inp_ref[...].astype(jnp.float32) + sw, 0.0)


def _tc_depth(inp, msg, gm, wht):
    return pl.pallas_call(
        _depth_body,
        grid=(_EG,),
        in_specs=[
            pl.BlockSpec((_EB, OUT_DIM), lambda i: (i, 0)),
            pl.BlockSpec((_EB, OUT_DIM), lambda i: (i, 0)),
            pl.BlockSpec((_EB, OUT_DIM), lambda i: (i, 0)),
            pl.BlockSpec((OUT_DIM, OUT_DIM), lambda i: (0, 0)),
        ],
        out_specs=pl.BlockSpec((_EB, OUT_DIM), lambda i: (i, 0)),
        out_shape=jax.ShapeDtypeStruct((N_EDGES, OUT_DIM), jnp.float32),
    )(inp, msg, gm, wht)


def _g_body(parts_ref, wh_ref, out_ref):
    p = parts_ref[...]
    out_ref[...] = jnp.dot(p[0] + p[1], wh_ref[...],
                           preferred_element_type=jnp.float32)


def _tc_g(parts, wht):
    return pl.pallas_call(
        _g_body,
        grid=(_NG,),
        in_specs=[
            pl.BlockSpec((NC, _NB, OUT_DIM), lambda i: (0, i, 0)),
            pl.BlockSpec((OUT_DIM, OUT_DIM), lambda i: (0, 0)),
        ],
        out_specs=pl.BlockSpec((_NB, OUT_DIM), lambda i: (i, 0)),
        out_shape=jax.ShapeDtypeStruct((N_NODES, OUT_DIM), jnp.float32),
    )(parts, wht)


def _final_body(x_ref, parts_ref, wox_ref, woo_ref, bo_ref, out_ref):
    p = parts_ref[...]
    h = jnp.dot(x_ref[...], wox_ref[...], preferred_element_type=jnp.float32)
    h += jnp.dot(p[0] + p[1], woo_ref[...], preferred_element_type=jnp.float32)
    out_ref[...] = jnp.maximum(h + bo_ref[...], 0.0)


def _tc_final(x, parts, woxt, woot, bo):
    return pl.pallas_call(
        _final_body,
        grid=(_NG,),
        in_specs=[
            pl.BlockSpec((_NB, ATOM_DIM), lambda i: (i, 0)),
            pl.BlockSpec((NC, _NB, OUT_DIM), lambda i: (0, i, 0)),
            pl.BlockSpec((ATOM_DIM, OUT_DIM), lambda i: (0, 0)),
            pl.BlockSpec((OUT_DIM, OUT_DIM), lambda i: (0, 0)),
            pl.BlockSpec((1, OUT_DIM), lambda i: (0, 0)),
        ],
        out_specs=pl.BlockSpec((_NB, OUT_DIM), lambda i: (i, 0)),
        out_shape=jax.ShapeDtypeStruct((N_NODES, OUT_DIM), jnp.float32),
    )(x, parts, woxt, woot, bo)


def kernel(x, edge_index, edge_attr, W_i, W_h, W_o, b_o):
    src = edge_index[0].astype(jnp.int32)
    dst = edge_index[1].astype(jnp.int32)
    src3 = src.reshape(G3, 2, HALF)
    dst3 = dst.reshape(G3, 2, HALF)
    dstw = dst.reshape(NW, NCHS, CHS)
    zeros = jnp.zeros((N_NODES, OUT_DIM), jnp.float32)

    wxt = W_i[:, :ATOM_DIM].T
    wbt = W_i[:, ATOM_DIM:].T
    wht = W_h.T
    woxt = W_o[:, :ATOM_DIM].T
    woot = W_o[:, ATOM_DIM:].T
    bo2 = b_o.reshape(1, OUT_DIM)

    xg = _sc_gather(x, src3)
    inp, msg = _tc_p0(xg, edge_attr, wxt, wbt)

    parts = _sc_scatter(msg, dstw, zeros)
    for _ in range(DEPTH - 1):
        g = _tc_g(parts, wht)
        gm = _sc_gather(g, dst3)
        msg = _tc_depth(inp, msg, gm, wht)
        parts = _sc_scatter(msg, dstw, zeros)

    return _tc_final(x, parts, woxt, woot, bo2)


# speedup vs baseline: 1.9501x; 1.0030x over previous
import jax
import jax.numpy as jnp
from jax import lax
from jax.experimental import pallas as pl
from jax.experimental.pallas import tpu as pltpu
from jax.experimental.pallas import tpu_sc as plsc

N_NODES = 10000
N_EDGES = 320000
ATOM_DIM = 128
OUT_DIM = 128
BOND_DIM = 16
DEPTH = 6

NC, NS = 2, 16
NW = NC * NS
EW = N_EDGES // NW
CH = 200
HALF = CH // 2
NCH = EW // CH
G3 = N_EDGES // CH
NPT = 624
NTAIL = N_NODES - NPT * NS

def _sc_mesh():
    return plsc.VectorSubcoreMesh(
        core_axis_name="c", subcore_axis_name="s", num_cores=NC, num_subcores=NS)


def _worker_id():
    return lax.axis_index("s") * NC + lax.axis_index("c")


NT = NCH // 2


def _sc_gather(table, idx3):
    nrow = table.shape[0]

    dt = table.dtype

    def body(table_ref, idx3_ref, out_ref, idxbuf, rowbuf, gs0, gs1, os0, os1):
        wid = _worker_id()
        pltpu.sync_copy(idx3_ref.at[pl.ds(wid * NCH, NCH)], idxbuf)

        def issue_gather(ci, b, sem):
            pltpu.async_copy(table_ref.at[idxbuf.at[ci, 0]],
                             rowbuf.at[b, pl.ds(0, HALF)], sem)
            pltpu.async_copy(table_ref.at[idxbuf.at[ci, 1]],
                             rowbuf.at[b, pl.ds(HALF, HALF)], sem)

        def wait_gather(b, sem):
            pltpu.make_async_copy(table_ref.at[pl.ds(0, CH)],
                                  rowbuf.at[b], sem).wait()

        def wait_store(b, sem):
            pltpu.make_async_copy(rowbuf.at[b],
                                  out_ref.at[pl.ds(0, CH)], sem).wait()

        issue_gather(0, 0, gs0)
        issue_gather(1, 1, gs1)

        def step(t, carry):
            c0 = 2 * t
            wait_gather(0, gs0)
            pltpu.async_copy(rowbuf.at[0],
                             out_ref.at[pl.ds((wid * NCH + c0) * CH, CH)], os0)
            wait_gather(1, gs1)
            pltpu.async_copy(rowbuf.at[1],
                             out_ref.at[pl.ds((wid * NCH + c0 + 1) * CH, CH)],
                             os1)

            @pl.when(t + 1 < NT)
            def _next():
                wait_store(0, os0)
                issue_gather(c0 + 2, 0, gs0)
                wait_store(1, os1)
                issue_gather(c0 + 3, 1, gs1)

            return carry

        lax.fori_loop(0, NT, step, 0)
        wait_store(0, os0)
        wait_store(1, os1)

    f = pl.kernel(
        body,
        out_type=jax.ShapeDtypeStruct((N_EDGES, OUT_DIM), dt),
        mesh=_sc_mesh(),
        scratch_types=[
            pltpu.VMEM((NCH, 2, HALF), jnp.int32),
            pltpu.VMEM((2, CH, OUT_DIM), dt),
            pltpu.SemaphoreType.DMA,
            pltpu.SemaphoreType.DMA,
            pltpu.SemaphoreType.DMA,
            pltpu.SemaphoreType.DMA,
        ],
        name=f"sc_gather_{nrow}",
    )
    return f(table, idx3)


CHS = 80
NCHS = EW // CHS
NTS = (NCHS + 1) // 2


def _sc_scatter(rows, idxw, zeros):
    def body(rows_ref, idxw_ref, zeros_ref, parts_ref, idxbuf, rowbuf, esum,
             ls0, ls1):
        cid = lax.axis_index("c")
        sid = lax.axis_index("s")
        wid = _worker_id()

        pltpu.sync_copy(idxw_ref.at[wid], idxbuf)

        def issue_load(ci, b, sem):
            pltpu.async_copy(rows_ref.at[pl.ds((wid * NCHS + ci) * CHS, CHS)],
                             rowbuf.at[b], sem)

        def wait_load(b, sem):
            pltpu.make_async_copy(rows_ref.at[pl.ds(0, CHS)],
                                  rowbuf.at[b], sem).wait()

        def scatter(ci, b):
            pltpu.sync_copy(rowbuf.at[b], esum.at[idxbuf.at[ci]], add=True)

        issue_load(0, 0, ls0)
        issue_load(1, 1, ls1)

        pltpu.sync_copy(zeros_ref.at[pl.ds(sid * NPT, NPT)],
                        esum.at[pl.ds(sid * NPT, NPT)])

        @pl.when(sid == 0)
        def _init_tail():
            pltpu.sync_copy(zeros_ref.at[pl.ds(NPT * NS, NTAIL)],
                            esum.at[pl.ds(NPT * NS, NTAIL)])

        plsc.subcore_barrier()

        def step(t, carry):
            c0 = 2 * t
            wait_load(0, ls0)
            scatter(c0, 0)

            @pl.when(c0 + 2 < NCHS)
            def _next0():
                issue_load(c0 + 2, 0, ls0)

            @pl.when(c0 + 1 < NCHS)
            def _odd():
                wait_load(1, ls1)
                scatter(c0 + 1, 1)

                @pl.when(c0 + 3 < NCHS)
                def _next1():
                    issue_load(c0 + 3, 1, ls1)

            return carry

        lax.fori_loop(0, NTS, step, 0)
        plsc.subcore_barrier()
        pltpu.sync_copy(esum.at[pl.ds(sid * NPT, NPT)],
                        parts_ref.at[cid, pl.ds(sid * NPT, NPT)])

        @pl.when(sid == 0)
        def _out_tail():
            pltpu.sync_copy(esum.at[pl.ds(NPT * NS, NTAIL)],
                            parts_ref.at[cid, pl.ds(NPT * NS, NTAIL)])

    f = pl.kernel(
        body,
        out_type=jax.ShapeDtypeStruct((NC, N_NODES, OUT_DIM), jnp.float32),
        mesh=_sc_mesh(),
        scratch_types=[
            pltpu.VMEM((NCHS, CHS), jnp.int32),
            pltpu.VMEM((2, CHS, OUT_DIM), jnp.float32),
            pltpu.VMEM_SHARED((N_NODES, OUT_DIM), jnp.float32),
            pltpu.SemaphoreType.DMA,
            pltpu.SemaphoreType.DMA,
        ],
        name="sc_scatter_add",
    )
    return f(rows, idxw, zeros)


_EB = 16000
_EG = N_EDGES // _EB
_NB = 1000
_NG = N_NODES // _NB


def _p0_body(xg_ref, ea_ref, wx_ref, wb_ref, inp_ref, msg_ref):
    v = jnp.dot(xg_ref[...], wx_ref[...], preferred_element_type=jnp.float32)
    v += jnp.dot(ea_ref[...], wb_ref[...], preferred_element_type=jnp.float32)
    inp_ref[...] = v.astype(inp_ref.dtype)
    msg_ref[...] = jnp.maximum(v, 0.0)


def _tc_p0(xg, ea, wxt, wbt):
    return pl.pallas_call(
        _p0_body,
        grid=(_EG,),
        in_specs=[
            pl.BlockSpec((_EB, ATOM_DIM), lambda i: (i, 0)),
            pl.BlockSpec((_EB, BOND_DIM), lambda i: (i, 0)),
            pl.BlockSpec((ATOM_DIM, OUT_DIM), lambda i: (0, 0)),
            pl.BlockSpec((BOND_DIM, OUT_DIM), lambda i: (0, 0)),
        ],
        out_specs=[
            pl.BlockSpec((_EB, OUT_DIM), lambda i: (i, 0)),
            pl.BlockSpec((_EB, OUT_DIM), lambda i: (i, 0)),
        ],
        out_shape=[
            jax.ShapeDtypeStruct((N_EDGES, OUT_DIM), jnp.bfloat16),
            jax.ShapeDtypeStruct((N_EDGES, OUT_DIM), jnp.float32),
        ],
    )(xg, ea, wxt, wbt)


def _swap_pairs(a3):
    return jnp.concatenate([a3[:, 1:2, :], a3[:, 0:1, :]], axis=1)


def _depth_body(inp_ref, msg_ref, gm_ref, wh_ref, out_ref):
    t = jnp.dot(msg_ref[...], wh_ref[...], preferred_element_type=jnp.float32)
    d3 = (gm_ref[...] - t).reshape(_EB // 2, 2, OUT_DIM)
    sw = _swap_pairs(d3).reshape(_EB, OUT_DIM)
    out_ref[...] = jnp.maximum(inp_ref[...].astype(jnp.float32) + sw, 0.0)


def _tc_depth(inp, msg, gm, wht):
    return pl.pallas_call(
        _depth_body,
        grid=(_EG,),
        in_specs=[
            pl.BlockSpec((_EB, OUT_DIM), lambda i: (i, 0)),
            pl.BlockSpec((_EB, OUT_DIM), lambda i: (i, 0)),
            pl.BlockSpec((_EB, OUT_DIM), lambda i: (i, 0)),
            pl.BlockSpec((OUT_DIM, OUT_DIM), lambda i: (0, 0)),
        ],
        out_specs=pl.BlockSpec((_EB, OUT_DIM), lambda i: (i, 0)),
        out_shape=jax.ShapeDtypeStruct((N_EDGES, OUT_DIM), jnp.float32),
    )(inp, msg, gm, wht)


def _g_body(parts_ref, wh_ref, out_ref):
    p = parts_ref[...]
    out_ref[...] = jnp.dot(p[0] + p[1], wh_ref[...],
                           preferred_element_type=jnp.float32)


def _tc_g(parts, wht):
    return pl.pallas_call(
        _g_body,
        grid=(_NG,),
        in_specs=[
            pl.BlockSpec((NC, _NB, OUT_DIM), lambda i: (0, i, 0)),
            pl.BlockSpec((OUT_DIM, OUT_DIM), lambda i: (0, 0)),
        ],
        out_specs=pl.BlockSpec((_NB, OUT_DIM), lambda i: (i, 0)),
        out_shape=jax.ShapeDtypeStruct((N_NODES, OUT_DIM), jnp.float32),
    )(parts, wht)


def _final_body(x_ref, parts_ref, wox_ref, woo_ref, bo_ref, out_ref):
    p = parts_ref[...]
    h = jnp.dot(x_ref[...], wox_ref[...], preferred_element_type=jnp.float32)
    h += jnp.dot(p[0] + p[1], woo_ref[...], preferred_element_type=jnp.float32)
    out_ref[...] = jnp.maximum(h + bo_ref[...], 0.0)


def _tc_final(x, parts, woxt, woot, bo):
    return pl.pallas_call(
        _final_body,
        grid=(_NG,),
        in_specs=[
            pl.BlockSpec((_NB, ATOM_DIM), lambda i: (i, 0)),
            pl.BlockSpec((NC, _NB, OUT_DIM), lambda i: (0, i, 0)),
            pl.BlockSpec((ATOM_DIM, OUT_DIM), lambda i: (0, 0)),
            pl.BlockSpec((OUT_DIM, OUT_DIM), lambda i: (0, 0)),
            pl.BlockSpec((1, OUT_DIM), lambda i: (0, 0)),
        ],
        out_specs=pl.BlockSpec((_NB, OUT_DIM), lambda i: (i, 0)),
        out_shape=jax.ShapeDtypeStruct((N_NODES, OUT_DIM), jnp.float32),
    )(x, parts, woxt, woot, bo)


def kernel(x, edge_index, edge_attr, W_i, W_h, W_o, b_o):
    src = edge_index[0].astype(jnp.int32)
    dst = edge_index[1].astype(jnp.int32)
    src3 = src.reshape(G3, 2, HALF)
    dst3 = dst.reshape(G3, 2, HALF)
    dstw = dst.reshape(NW, NCHS, CHS)
    zeros = jnp.zeros((N_NODES, OUT_DIM), jnp.float32)

    wxt = W_i[:, :ATOM_DIM].T
    wbt = W_i[:, ATOM_DIM:].T
    wht = W_h.T
    woxt = W_o[:, :ATOM_DIM].T
    woot = W_o[:, ATOM_DIM:].T
    bo2 = b_o.reshape(1, OUT_DIM)

    xg = _sc_gather(x, src3)
    inp, msg = _tc_p0(xg, edge_attr, wxt, wbt)

    parts = _sc_scatter(msg, dstw, zeros)
    for _ in range(DEPTH - 1):
        g = _tc_g(parts, wht)
        gm = _sc_gather(g, dst3)
        msg = _tc_depth(inp, msg, gm, wht)
        parts = _sc_scatter(msg, dstw, zeros)

    return _tc_final(x, parts, woxt, woot, bo2)
